# Initial kernel scaffold; baseline (speedup 1.0000x reference)
import jax
import jax.numpy as jnp
from jax.experimental import pallas as pl


def kernel(heatmap):
    raise NotImplementedError("write your pallas kernel here")


# speedup vs baseline: 1.5791x; 1.5791x over previous
import functools

import jax
import jax.numpy as jnp
from jax.experimental import pallas as pl

MAX_FLARES = 6
NMS_KERNEL = 9
CONF_THRESHOLD = 0.3


def _poshead_kernel(hm_ref, out_ref):
    H, W = hm_ref.shape[1], hm_ref.shape[2]
    hm = hm_ref[0]
    pad = NMS_KERNEL // 2
    neg = jnp.float32(-jnp.inf)

    pr = jnp.concatenate(
        [jnp.full((pad, W), neg, jnp.float32), hm, jnp.full((pad, W), neg, jnp.float32)],
        axis=0,
    )
    rowmax = hm
    for dy in range(NMS_KERNEL):
        if dy == pad:
            continue
        rowmax = jnp.maximum(rowmax, jax.lax.slice(pr, (dy, 0), (dy + H, W)))
    pc = jnp.concatenate(
        [jnp.full((H, pad), neg, jnp.float32), rowmax, jnp.full((H, pad), neg, jnp.float32)],
        axis=1,
    )
    pooled = rowmax
    for dx in range(NMS_KERNEL):
        if dx == pad:
            continue
        pooled = jnp.maximum(pooled, jax.lax.slice(pc, (0, dx), (H, dx + W)))

    peaks = (hm == pooled) & (hm > CONF_THRESHOLD)
    scores = jnp.where(peaks, hm, 0.0)

    lin = (
        jax.lax.broadcasted_iota(jnp.int32, (H, W), 0) * W
        + jax.lax.broadcasted_iota(jnp.int32, (H, W), 1)
    )
    big = jnp.int32(2**30)
    inv_step = jnp.float32(1.0) / jnp.float32(W - 1)

    rix = jax.lax.broadcasted_iota(jnp.int32, (8, 128), 0)
    cix = jax.lax.broadcasted_iota(jnp.int32, (8, 128), 1)
    out = jnp.zeros((8, 128), jnp.float32)
    for k in range(MAX_FLARES):
        m = jnp.max(scores)
        idx = jnp.min(jnp.where(scores == m, lin, big))
        valid = m >= CONF_THRESHOLD
        x = jnp.where(valid, (idx % W).astype(jnp.float32) * inv_step, -1.0)
        y = jnp.where(valid, (idx // W).astype(jnp.float32) * inv_step, -1.0)
        conf = jnp.where(valid, m, 0.0)
        sel = cix == k
        out = jnp.where((rix == 0) & sel, conf, out)
        out = jnp.where((rix == 1) & sel, x, out)
        out = jnp.where((rix == 2) & sel, y, out)
        if k + 1 < MAX_FLARES:
            scores = jnp.where(lin == idx, 0.0, scores)
    out_ref[0] = out


@jax.jit
def kernel(heatmap):
    B, _, H, W = heatmap.shape
    hm = heatmap.reshape(B, H, W)
    out = pl.pallas_call(
        _poshead_kernel,
        grid=(B,),
        in_specs=[pl.BlockSpec((1, H, W), lambda b: (b, 0, 0))],
        out_specs=pl.BlockSpec((1, 8, 128), lambda b: (b, 0, 0)),
        out_shape=jax.ShapeDtypeStruct((B, 8, 128), jnp.float32),
    )(hm)
    conf = out[:, 0, :MAX_FLARES]
    pos = jnp.stack([out[:, 1, :MAX_FLARES], out[:, 2, :MAX_FLARES]], axis=-1)
    return pos, conf


# speedup vs baseline: 2.5505x; 1.6152x over previous
import jax
import jax.numpy as jnp
from jax.experimental import pallas as pl
from jax.experimental.pallas import tpu as pltpu

MAX_FLARES = 6
NMS_KERNEL = 9
CONF_THRESHOLD = 0.3


def _poshead_kernel(hm_ref, out_ref, sc_ref):
    H, W = hm_ref.shape[1], hm_ref.shape[2]
    hm = hm_ref[0]
    pad = NMS_KERNEL // 2
    neg = jnp.float32(-jnp.inf)

    xp = jnp.concatenate(
        [jnp.full((pad, W), neg, jnp.float32), hm, jnp.full((pad, W), neg, jnp.float32)],
        axis=0,
    )
    s1 = jnp.maximum(jax.lax.slice(xp, (0, 0), (H + 7, W)),
                     jax.lax.slice(xp, (1, 0), (H + 8, W)))
    s2 = jnp.maximum(jax.lax.slice(s1, (0, 0), (H + 5, W)),
                     jax.lax.slice(s1, (2, 0), (H + 7, W)))
    s3 = jnp.maximum(jax.lax.slice(s2, (0, 0), (H + 1, W)),
                     jax.lax.slice(s2, (4, 0), (H + 5, W)))
    rowp = jnp.maximum(jax.lax.slice(s3, (0, 0), (H, W)),
                       jax.lax.slice(xp, (8, 0), (H + 8, W)))

    yp = jnp.concatenate(
        [jnp.full((H, pad), neg, jnp.float32), rowp, jnp.full((H, pad), neg, jnp.float32)],
        axis=1,
    )
    t1 = jnp.maximum(jax.lax.slice(yp, (0, 0), (H, W + 7)),
                     jax.lax.slice(yp, (0, 1), (H, W + 8)))
    t2 = jnp.maximum(jax.lax.slice(t1, (0, 0), (H, W + 5)),
                     jax.lax.slice(t1, (0, 2), (H, W + 7)))
    t3 = jnp.maximum(jax.lax.slice(t2, (0, 0), (H, W + 1)),
                     jax.lax.slice(t2, (0, 4), (H, W + 5)))
    pooled = jnp.maximum(jax.lax.slice(t3, (0, 0), (H, W)),
                         jax.lax.slice(yp, (0, 8), (H, W + 8)))

    peaks = (hm == pooled) & (hm > CONF_THRESHOLD)
    scores = jnp.where(peaks, hm, 0.0)
    sc_ref[...] = scores
    rowmax = jnp.max(scores, axis=1, keepdims=True)

    riota = jax.lax.broadcasted_iota(jnp.int32, (H, 1), 0)
    ciota = jax.lax.broadcasted_iota(jnp.int32, (1, W), 1)
    big = jnp.int32(2**30)
    inv_step = jnp.float32(1.0) / jnp.float32(W - 1)

    rix = jax.lax.broadcasted_iota(jnp.int32, (8, 128), 0)
    cix = jax.lax.broadcasted_iota(jnp.int32, (8, 128), 1)
    out = jnp.zeros((8, 128), jnp.float32)
    for k in range(MAX_FLARES):
        m = jnp.max(rowmax)
        r = jnp.min(jnp.where(rowmax == m, riota, big))
        row = sc_ref[pl.ds(r, 1), :]
        c = jnp.min(jnp.where(row == m, ciota, big))
        valid = m >= CONF_THRESHOLD
        x = jnp.where(valid, c.astype(jnp.float32) * inv_step, -1.0)
        y = jnp.where(valid, r.astype(jnp.float32) * inv_step, -1.0)
        conf = jnp.where(valid, m, 0.0)
        sel = cix == k
        out = jnp.where((rix == 0) & sel, conf, out)
        out = jnp.where((rix == 1) & sel, x, out)
        out = jnp.where((rix == 2) & sel, y, out)
        if k + 1 < MAX_FLARES:
            newrow = jnp.where(ciota == c, 0.0, row)
            sc_ref[pl.ds(r, 1), :] = newrow
            rowmax = jnp.where(riota == r, jnp.max(newrow), rowmax)
    out_ref[0] = out


@jax.jit
def kernel(heatmap):
    B, _, H, W = heatmap.shape
    hm = heatmap.reshape(B, H, W)
    out = pl.pallas_call(
        _poshead_kernel,
        grid=(B,),
        in_specs=[pl.BlockSpec((1, H, W), lambda b: (b, 0, 0))],
        out_specs=pl.BlockSpec((1, 8, 128), lambda b: (b, 0, 0)),
        out_shape=jax.ShapeDtypeStruct((B, 8, 128), jnp.float32),
        scratch_shapes=[pltpu.VMEM((H, W), jnp.float32)],
    )(hm)
    conf = out[:, 0, :MAX_FLARES]
    pos = jnp.stack([out[:, 1, :MAX_FLARES], out[:, 2, :MAX_FLARES]], axis=-1)
    return pos, conf


# speedup vs baseline: 2.8246x; 1.1074x over previous
import jax
import jax.numpy as jnp
from jax.experimental import pallas as pl
from jax.experimental.pallas import tpu as pltpu

MAX_FLARES = 6
NMS_KERNEL = 9
CONF_THRESHOLD = 0.3


def _poshead_one(hm, sc_ref):
    H, W = hm.shape
    pad = NMS_KERNEL // 2
    neg = jnp.float32(-jnp.inf)

    xp = jnp.concatenate(
        [jnp.full((pad, W), neg, jnp.float32), hm, jnp.full((pad, W), neg, jnp.float32)],
        axis=0,
    )
    s1 = jnp.maximum(jax.lax.slice(xp, (0, 0), (H + 7, W)),
                     jax.lax.slice(xp, (1, 0), (H + 8, W)))
    s2 = jnp.maximum(jax.lax.slice(s1, (0, 0), (H + 5, W)),
                     jax.lax.slice(s1, (2, 0), (H + 7, W)))
    s3 = jnp.maximum(jax.lax.slice(s2, (0, 0), (H + 1, W)),
                     jax.lax.slice(s2, (4, 0), (H + 5, W)))
    rowp = jnp.maximum(jax.lax.slice(s3, (0, 0), (H, W)),
                       jax.lax.slice(xp, (8, 0), (H + 8, W)))

    yp = jnp.concatenate(
        [jnp.full((H, pad), neg, jnp.float32), rowp, jnp.full((H, pad), neg, jnp.float32)],
        axis=1,
    )
    t1 = jnp.maximum(jax.lax.slice(yp, (0, 0), (H, W + 7)),
                     jax.lax.slice(yp, (0, 1), (H, W + 8)))
    t2 = jnp.maximum(jax.lax.slice(t1, (0, 0), (H, W + 5)),
                     jax.lax.slice(t1, (0, 2), (H, W + 7)))
    t3 = jnp.maximum(jax.lax.slice(t2, (0, 0), (H, W + 1)),
                     jax.lax.slice(t2, (0, 4), (H, W + 5)))
    pooled = jnp.maximum(jax.lax.slice(t3, (0, 0), (H, W)),
                         jax.lax.slice(yp, (0, 8), (H, W + 8)))

    peaks = (hm == pooled) & (hm > CONF_THRESHOLD)
    scores = jnp.where(peaks, hm, 0.0)
    sc_ref[...] = scores
    rowmax = jnp.max(scores, axis=1, keepdims=True)

    riota = jax.lax.broadcasted_iota(jnp.int32, (H, 1), 0)
    ciota = jax.lax.broadcasted_iota(jnp.int32, (1, W), 1)
    big = jnp.int32(2**30)
    inv_step = jnp.float32(1.0) / jnp.float32(W - 1)

    rix = jax.lax.broadcasted_iota(jnp.int32, (8, 128), 0)
    cix = jax.lax.broadcasted_iota(jnp.int32, (8, 128), 1)
    out = jnp.zeros((8, 128), jnp.float32)
    for k in range(MAX_FLARES):
        m = jnp.max(rowmax)
        r = jnp.min(jnp.where(rowmax == m, riota, big))
        row = sc_ref[pl.ds(r, 1), :]
        c = jnp.min(jnp.where(row == m, ciota, big))
        valid = m >= CONF_THRESHOLD
        x = jnp.where(valid, c.astype(jnp.float32) * inv_step, -1.0)
        y = jnp.where(valid, r.astype(jnp.float32) * inv_step, -1.0)
        conf = jnp.where(valid, m, 0.0)
        sel = cix == k
        out = jnp.where((rix == 0) & sel, conf, out)
        out = jnp.where((rix == 1) & sel, x, out)
        out = jnp.where((rix == 2) & sel, y, out)
        if k + 1 < MAX_FLARES:
            newrow = jnp.where(ciota == c, 0.0, row)
            sc_ref[pl.ds(r, 1), :] = newrow
            rowmax = jnp.where(riota == r, jnp.max(newrow), rowmax)
    return out


N_PER = 2


def _poshead_kernel(hm_ref, out_ref, sc_ref):
    for j in range(N_PER):
        out_ref[j] = _poshead_one(hm_ref[j], sc_ref.at[j])


@jax.jit
def kernel(heatmap):
    B, _, H, W = heatmap.shape
    hm = heatmap.reshape(B, H, W)
    out = pl.pallas_call(
        _poshead_kernel,
        grid=(B // N_PER,),
        in_specs=[pl.BlockSpec((N_PER, H, W), lambda b: (b, 0, 0))],
        out_specs=pl.BlockSpec((N_PER, 8, 128), lambda b: (b, 0, 0)),
        out_shape=jax.ShapeDtypeStruct((B, 8, 128), jnp.float32),
        scratch_shapes=[pltpu.VMEM((N_PER, H, W), jnp.float32)],
    )(hm)
    conf = out[:, 0, :MAX_FLARES]
    pos = jnp.stack([out[:, 1, :MAX_FLARES], out[:, 2, :MAX_FLARES]], axis=-1)
    return pos, conf


# speedup vs baseline: 3.0988x; 1.0971x over previous
import jax
import jax.numpy as jnp
from jax.experimental import pallas as pl
from jax.experimental.pallas import tpu as pltpu

MAX_FLARES = 6
NMS_KERNEL = 9
CONF_THRESHOLD = 0.3


def _poshead_one(hm, sc_ref):
    H, W = hm.shape
    pad = NMS_KERNEL // 2
    neg = jnp.float32(-jnp.inf)

    xp = jnp.concatenate(
        [jnp.full((pad, W), neg, jnp.float32), hm, jnp.full((pad, W), neg, jnp.float32)],
        axis=0,
    )
    s1 = jnp.maximum(jax.lax.slice(xp, (0, 0), (H + 7, W)),
                     jax.lax.slice(xp, (1, 0), (H + 8, W)))
    s2 = jnp.maximum(jax.lax.slice(s1, (0, 0), (H + 5, W)),
                     jax.lax.slice(s1, (2, 0), (H + 7, W)))
    s3 = jnp.maximum(jax.lax.slice(s2, (0, 0), (H + 1, W)),
                     jax.lax.slice(s2, (4, 0), (H + 5, W)))
    rowp = jnp.maximum(jax.lax.slice(s3, (0, 0), (H, W)),
                       jax.lax.slice(xp, (8, 0), (H + 8, W)))

    yp = jnp.concatenate(
        [jnp.full((H, pad), neg, jnp.float32), rowp, jnp.full((H, pad), neg, jnp.float32)],
        axis=1,
    )
    t1 = jnp.maximum(jax.lax.slice(yp, (0, 0), (H, W + 7)),
                     jax.lax.slice(yp, (0, 1), (H, W + 8)))
    t2 = jnp.maximum(jax.lax.slice(t1, (0, 0), (H, W + 5)),
                     jax.lax.slice(t1, (0, 2), (H, W + 7)))
    t3 = jnp.maximum(jax.lax.slice(t2, (0, 0), (H, W + 1)),
                     jax.lax.slice(t2, (0, 4), (H, W + 5)))
    pooled = jnp.maximum(jax.lax.slice(t3, (0, 0), (H, W)),
                         jax.lax.slice(yp, (0, 8), (H, W + 8)))

    peaks = (hm == pooled) & (hm > CONF_THRESHOLD)
    scores = jnp.where(peaks, hm, 0.0)
    sc_ref[...] = scores
    rowmax = jnp.max(scores, axis=1, keepdims=True)

    riota = jax.lax.broadcasted_iota(jnp.int32, (H, 1), 0)
    ciota = jax.lax.broadcasted_iota(jnp.int32, (1, W), 1)
    big = jnp.int32(2**30)
    inv_step = jnp.float32(1.0) / jnp.float32(W - 1)

    rix = jax.lax.broadcasted_iota(jnp.int32, (8, 128), 0)
    cix = jax.lax.broadcasted_iota(jnp.int32, (8, 128), 1)
    out = jnp.zeros((8, 128), jnp.float32)
    for k in range(MAX_FLARES):
        m = jnp.max(rowmax)
        r = jnp.min(jnp.where(rowmax == m, riota, big))
        row = sc_ref[pl.ds(r, 1), :]
        c = jnp.min(jnp.where(row == m, ciota, big))
        valid = m >= CONF_THRESHOLD
        x = jnp.where(valid, c.astype(jnp.float32) * inv_step, -1.0)
        y = jnp.where(valid, r.astype(jnp.float32) * inv_step, -1.0)
        conf = jnp.where(valid, m, 0.0)
        sel = cix == k
        out = jnp.where((rix == 0) & sel, conf, out)
        out = jnp.where((rix == 1) & sel, x, out)
        out = jnp.where((rix == 2) & sel, y, out)
        if k + 1 < MAX_FLARES:
            newrow = jnp.where(ciota == c, 0.0, row)
            sc_ref[pl.ds(r, 1), :] = newrow
            rowmax = jnp.where(riota == r, jnp.max(newrow), rowmax)
    return out


N_PER = 4


def _poshead_kernel(hm_ref, out_ref, sc_ref):
    for j in range(N_PER):
        out_ref[j] = _poshead_one(hm_ref[j], sc_ref.at[j])


@jax.jit
def kernel(heatmap):
    B, _, H, W = heatmap.shape
    hm = heatmap.reshape(B, H, W)
    out = pl.pallas_call(
        _poshead_kernel,
        grid=(B // N_PER,),
        in_specs=[pl.BlockSpec((N_PER, H, W), lambda b: (b, 0, 0))],
        out_specs=pl.BlockSpec((N_PER, 8, 128), lambda b: (b, 0, 0)),
        out_shape=jax.ShapeDtypeStruct((B, 8, 128), jnp.float32),
        scratch_shapes=[pltpu.VMEM((N_PER, H, W), jnp.float32)],
    )(hm)
    conf = out[:, 0, :MAX_FLARES]
    pos = jnp.stack([out[:, 1, :MAX_FLARES], out[:, 2, :MAX_FLARES]], axis=-1)
    return pos, conf
